# Initial kernel scaffold; baseline (speedup 1.0000x reference)
import functools

import jax
import jax.numpy as jnp
from jax import lax
from jax.experimental import pallas as pl
from jax.experimental.pallas import tpu as pltpu
from jax.experimental.pallas import tpu_sc as plsc

N = 10000
E = 320000
D = 128
G = 128

NC = 2
NS = 16
CHUNK = 128
CH = 80
EP = NC * NS * CH * CHUNK
NP = 10240
RPT = NP // NS


def _sc_aggregate(feat, src_t, dst_t, zeros, dw):
    mesh = plsc.VectorSubcoreMesh(core_axis_name="c", subcore_axis_name="s")

    @functools.partial(
        pl.kernel,
        out_type=jax.ShapeDtypeStruct((NC, NP, dw), jnp.float32),
        mesh=mesh,
        scratch_types=[
            pltpu.VMEM((CH, CHUNK), jnp.int32),
            pltpu.VMEM((CH, CHUNK), jnp.int32),
            pltpu.VMEM((CHUNK, dw), jnp.float32),
            pltpu.VMEM_SHARED((NP, dw), jnp.float32),
            pltpu.SemaphoreType.DMA,
        ],
    )
    def agg(feat_hbm, src_hbm, dst_hbm, zer_hbm, out_hbm,
            src_v, dst_v, rows_v, acc_s, sem):
        c = lax.axis_index("c")
        s = lax.axis_index("s")
        pltpu.sync_copy(zer_hbm.at[pl.ds(s * RPT, RPT)],
                        acc_s.at[pl.ds(s * RPT, RPT)])
        pltpu.sync_copy(src_hbm.at[c, s], src_v)
        pltpu.sync_copy(dst_hbm.at[c, s], dst_v)
        plsc.subcore_barrier()

        def body(i, carry):
            pltpu.async_copy(feat_hbm.at[src_v.at[i]], rows_v, sem).wait()
            pltpu.sync_copy(rows_v, acc_s.at[dst_v.at[i]], add=True)
            return carry

        lax.fori_loop(0, CH, body, 0, unroll=False)
        plsc.subcore_barrier()
        pltpu.sync_copy(acc_s.at[pl.ds(s * RPT, RPT)],
                        out_hbm.at[c, pl.ds(s * RPT, RPT)])

    return agg(feat, src_t, dst_t, zeros)


def _tc_layer1(parts, g1W1, g1b1, g2W1, g2b1):
    R = 1024
    grid = (NP // R,)
    dwa = D + 16

    def body(p_ref, w1_ref, b1_ref, w2_ref, b2_ref,
             h1_ref, h2_ref, ninv_ref):
        i = pl.program_id(0)
        S = p_ref[0] + p_ref[1]
        deg = S[:, D:D + 1]
        ninv = 1.0 / jnp.maximum(deg, 1.0)
        A = S[:, :D] * ninv
        rows = i * R + lax.broadcasted_iota(jnp.int32, (R, D), 0)
        valid = (rows < N).astype(jnp.float32)
        h1 = jnp.maximum(
            0.8 * jnp.dot(A, w1_ref[...], preferred_element_type=jnp.float32)
            + b1_ref[...], 0.0)
        h2 = jnp.maximum(
            0.9 * jnp.dot(A, w2_ref[...], preferred_element_type=jnp.float32)
            + b2_ref[...], 0.0)
        h1_ref[...] = h1 * valid
        h2_ref[...] = h2 * valid
        ninv_ref[...] = jnp.broadcast_to(ninv, (R, D))

    out = pl.pallas_call(
        body,
        grid=grid,
        in_specs=[
            pl.BlockSpec((NC, R, dwa), lambda i: (0, i, 0)),
            pl.BlockSpec((D, D), lambda i: (0, 0)),
            pl.BlockSpec((1, D), lambda i: (0, 0)),
            pl.BlockSpec((D, D), lambda i: (0, 0)),
            pl.BlockSpec((1, D), lambda i: (0, 0)),
        ],
        out_specs=[
            pl.BlockSpec((R, D), lambda i: (i, 0)),
            pl.BlockSpec((R, D), lambda i: (i, 0)),
            pl.BlockSpec((R, D), lambda i: (i, 0)),
        ],
        out_shape=[
            jax.ShapeDtypeStruct((NP, D), jnp.float32),
            jax.ShapeDtypeStruct((NP, D), jnp.float32),
            jax.ShapeDtypeStruct((NP, D), jnp.float32),
        ],
    )(parts, g1W1, g1b1, g2W1, g2b1)
    return out


def _tc_layer2_heads(p1, p2, ninv, batch_bc,
                     g1W2, g1b2, g2W2, g2b2,
                     m1W1, m1b1, m1W2, m1b2,
                     m2W1, m2b1, m2W2, m2b2):
    R = 1024
    grid = (NP // R,)
    nsteps = NP // R

    def body(p1_ref, p2_ref, ninv_ref, bt_ref,
             gw1_ref, gb1_ref, gw2_ref, gb2_ref,
             mw1_ref, mb1_ref, mw2_ref, mb2_ref,
             nw1_ref, nb1_ref, nw2_ref, nb2_ref,
             h1_ref, h2_ref, g1_ref, g2_ref,
             ps1_acc, ps2_acc, cnt_acc):
        i = pl.program_id(0)
        ninv = ninv_ref[...]
        S1 = p1_ref[0] + p1_ref[1]
        S2 = p2_ref[0] + p2_ref[1]
        z1 = jnp.maximum(
            jnp.dot(S1 * ninv, gw1_ref[...],
                    preferred_element_type=jnp.float32) + gb1_ref[...], 0.0)
        z2 = jnp.maximum(
            jnp.dot(S2 * ninv, gw2_ref[...],
                    preferred_element_type=jnp.float32) + gb2_ref[...], 0.0)
        t1 = jnp.maximum(
            jnp.dot(z1, mw1_ref[...], preferred_element_type=jnp.float32)
            + mb1_ref[...], 0.0)
        t2 = jnp.maximum(
            jnp.dot(z2, mw1_ref[...], preferred_element_type=jnp.float32)
            + mb1_ref[...], 0.0)
        h1_ref[...] = jnp.dot(t1, mw2_ref[...],
                              preferred_element_type=jnp.float32) + mb2_ref[...]
        h2_ref[...] = jnp.dot(t2, mw2_ref[...],
                              preferred_element_type=jnp.float32) + mb2_ref[...]
        rows = i * R + lax.broadcasted_iota(jnp.int32, (R, G), 0)
        valid = (rows < N).astype(jnp.float32)
        oh = (bt_ref[...] == lax.broadcasted_iota(jnp.int32, (R, G), 1))
        oh = oh.astype(jnp.float32)
        dn = (((0,), (0,)), ((), ()))

        @pl.when(i == 0)
        def _():
            ps1_acc[...] = jnp.zeros((G, D), jnp.float32)
            ps2_acc[...] = jnp.zeros((G, D), jnp.float32)
            cnt_acc[...] = jnp.zeros((G, D), jnp.float32)

        ps1_acc[...] += lax.dot_general(oh, z1 * valid, dn,
                                        preferred_element_type=jnp.float32)
        ps2_acc[...] += lax.dot_general(oh, z2 * valid, dn,
                                        preferred_element_type=jnp.float32)
        cnt_acc[...] += lax.dot_general(oh, valid, dn,
                                        preferred_element_type=jnp.float32)

        @pl.when(i == nsteps - 1)
        def _():
            icnt = 1.0 / jnp.maximum(cnt_acc[...], 1.0)
            gp1 = ps1_acc[...] * icnt
            gp2 = ps2_acc[...] * icnt
            u1 = jnp.maximum(
                jnp.dot(gp1, nw1_ref[...], preferred_element_type=jnp.float32)
                + nb1_ref[...], 0.0)
            u2 = jnp.maximum(
                jnp.dot(gp2, nw1_ref[...], preferred_element_type=jnp.float32)
                + nb1_ref[...], 0.0)
            g1_ref[...] = jnp.dot(u1, nw2_ref[...],
                                  preferred_element_type=jnp.float32) + nb2_ref[...]
            g2_ref[...] = jnp.dot(u2, nw2_ref[...],
                                  preferred_element_type=jnp.float32) + nb2_ref[...]

    wspec = pl.BlockSpec((D, D), lambda i: (0, 0))
    bspec = pl.BlockSpec((1, D), lambda i: (0, 0))
    out = pl.pallas_call(
        body,
        grid=grid,
        in_specs=[
            pl.BlockSpec((NC, R, D), lambda i: (0, i, 0)),
            pl.BlockSpec((NC, R, D), lambda i: (0, i, 0)),
            pl.BlockSpec((R, D), lambda i: (i, 0)),
            pl.BlockSpec((R, G), lambda i: (i, 0)),
            wspec, bspec, wspec, bspec,
            wspec, bspec, wspec, bspec,
            wspec, bspec, wspec, bspec,
        ],
        out_specs=[
            pl.BlockSpec((R, D), lambda i: (i, 0)),
            pl.BlockSpec((R, D), lambda i: (i, 0)),
            pl.BlockSpec((G, D), lambda i: (0, 0)),
            pl.BlockSpec((G, D), lambda i: (0, 0)),
        ],
        out_shape=[
            jax.ShapeDtypeStruct((NP, D), jnp.float32),
            jax.ShapeDtypeStruct((NP, D), jnp.float32),
            jax.ShapeDtypeStruct((G, D), jnp.float32),
            jax.ShapeDtypeStruct((G, D), jnp.float32),
        ],
        scratch_shapes=[
            pltpu.VMEM((G, D), jnp.float32),
            pltpu.VMEM((G, D), jnp.float32),
            pltpu.VMEM((G, D), jnp.float32),
        ],
    )(p1, p2, ninv, batch_bc,
      g1W2, g1b2, g2W2, g2b2,
      m1W1, m1b1, m1W2, m1b2,
      m2W1, m2b1, m2W2, m2b2)
    return out


def kernel(x, edge_index, batch,
           g1W1, g1b1, g1W2, g1b2,
           g2W1, g2b1, g2W2, g2b2,
           m1W1, m1b1, m1W2, m1b2,
           m2W1, m2b1, m2W2, m2b2):
    src = edge_index[0].astype(jnp.int32)
    dst = edge_index[1].astype(jnp.int32)
    pad = jnp.full((EP - E,), N, jnp.int32)
    src_t = jnp.concatenate([src, pad]).reshape(NC, NS, CH, CHUNK)
    dst_t = jnp.concatenate([dst, pad]).reshape(NC, NS, CH, CHUNK)

    dwa = D + 16
    xp = jnp.zeros((NP, dwa), jnp.float32)
    xp = xp.at[:N, :D].set(x).at[:N, D].set(1.0)
    zeros_a = jnp.zeros((NP, dwa), jnp.float32)
    zeros_d = jnp.zeros((NP, D), jnp.float32)

    parts0 = _sc_aggregate(xp, src_t, dst_t, zeros_a, dwa)
    h1, h2, ninv = _tc_layer1(parts0, g1W1, g1b1.reshape(1, D),
                              g2W1, g2b1.reshape(1, D))

    parts1 = _sc_aggregate(h1, src_t, dst_t, zeros_d, D)
    parts2 = _sc_aggregate(h2, src_t, dst_t, zeros_d, D)

    batch_bc = jnp.zeros((NP,), jnp.int32).at[:N].set(batch.astype(jnp.int32))
    batch_bc = jnp.broadcast_to(batch_bc[:, None], (NP, G))

    h1o, h2o, g1o, g2o = _tc_layer2_heads(
        parts1, parts2, ninv, batch_bc,
        g1W2, g1b2.reshape(1, D), g2W2, g2b2.reshape(1, D),
        m1W1, m1b1.reshape(1, D), m1W2, m1b2.reshape(1, D),
        m2W1, m2b1.reshape(1, D), m2W2, m2b2.reshape(1, D))

    return h1o[:N], h2o[:N], g1o, g2o


# speedup vs baseline: 2.5338x; 2.5338x over previous
import functools

import jax
import jax.numpy as jnp
from jax import lax
from jax.experimental import pallas as pl
from jax.experimental.pallas import tpu as pltpu
from jax.experimental.pallas import tpu_sc as plsc

N = 10000
E = 320000
D = 128
G = 128

NC = 2
NS = 16
CHUNK = 128
CH = 80
EP = NC * NS * CH * CHUNK
NP = 10240
RPT = NP // NS


def _sc_aggregate(feat, src_t, dst_t, zeros, dw):
    mesh = plsc.VectorSubcoreMesh(core_axis_name="c", subcore_axis_name="s")

    @functools.partial(
        pl.kernel,
        out_type=jax.ShapeDtypeStruct((NC, NP, dw), jnp.float32),
        mesh=mesh,
        compiler_params=pltpu.CompilerParams(use_tc_tiling_on_sc=False),
        scratch_types=[
            pltpu.VMEM((CH, CHUNK), jnp.int32),
            pltpu.VMEM((CH, CHUNK), jnp.int32),
            pltpu.VMEM((CHUNK, dw), jnp.float32),
            pltpu.VMEM_SHARED((NP, dw), jnp.float32),
            pltpu.SemaphoreType.DMA,
        ],
    )
    def agg(feat_hbm, src_hbm, dst_hbm, zer_hbm, out_hbm,
            src_v, dst_v, rows_v, acc_s, sem):
        c = lax.axis_index("c")
        s = lax.axis_index("s")
        pltpu.sync_copy(zer_hbm.at[pl.ds(s * RPT, RPT)],
                        acc_s.at[pl.ds(s * RPT, RPT)])
        pltpu.sync_copy(src_hbm.at[c, s], src_v)
        pltpu.sync_copy(dst_hbm.at[c, s], dst_v)
        plsc.subcore_barrier()

        def body(i, carry):
            pltpu.async_copy(feat_hbm.at[src_v.at[i]], rows_v, sem).wait()
            pltpu.sync_copy(rows_v, acc_s.at[dst_v.at[i]], add=True)
            return carry

        lax.fori_loop(0, CH, body, 0, unroll=False)
        plsc.subcore_barrier()
        pltpu.sync_copy(acc_s.at[pl.ds(s * RPT, RPT)],
                        out_hbm.at[c, pl.ds(s * RPT, RPT)])

    return agg(feat, src_t, dst_t, zeros)


def _tc_layer1(parts, g1W1, g1b1, g2W1, g2b1):
    R = 1024
    grid = (NP // R,)
    dwa = D + 16

    def body(p_ref, w1_ref, b1_ref, w2_ref, b2_ref,
             h1_ref, h2_ref, ninv_ref):
        i = pl.program_id(0)
        S = p_ref[0] + p_ref[1]
        deg = S[:, D:D + 1]
        ninv = 1.0 / jnp.maximum(deg, 1.0)
        A = S[:, :D] * ninv
        rows = i * R + lax.broadcasted_iota(jnp.int32, (R, D), 0)
        valid = (rows < N).astype(jnp.float32)
        h1 = jnp.maximum(
            0.8 * jnp.dot(A, w1_ref[...], preferred_element_type=jnp.float32)
            + b1_ref[...], 0.0)
        h2 = jnp.maximum(
            0.9 * jnp.dot(A, w2_ref[...], preferred_element_type=jnp.float32)
            + b2_ref[...], 0.0)
        h1_ref[...] = h1 * valid
        h2_ref[...] = h2 * valid
        ninv_ref[...] = jnp.broadcast_to(ninv, (R, D))

    out = pl.pallas_call(
        body,
        grid=grid,
        in_specs=[
            pl.BlockSpec((NC, R, dwa), lambda i: (0, i, 0)),
            pl.BlockSpec((D, D), lambda i: (0, 0)),
            pl.BlockSpec((1, D), lambda i: (0, 0)),
            pl.BlockSpec((D, D), lambda i: (0, 0)),
            pl.BlockSpec((1, D), lambda i: (0, 0)),
        ],
        out_specs=[
            pl.BlockSpec((R, D), lambda i: (i, 0)),
            pl.BlockSpec((R, D), lambda i: (i, 0)),
            pl.BlockSpec((R, D), lambda i: (i, 0)),
        ],
        out_shape=[
            jax.ShapeDtypeStruct((NP, D), jnp.float32),
            jax.ShapeDtypeStruct((NP, D), jnp.float32),
            jax.ShapeDtypeStruct((NP, D), jnp.float32),
        ],
    )(parts, g1W1, g1b1, g2W1, g2b1)
    return out


def _tc_layer2_heads(p1, p2, ninv, batch_bc,
                     g1W2, g1b2, g2W2, g2b2,
                     m1W1, m1b1, m1W2, m1b2,
                     m2W1, m2b1, m2W2, m2b2):
    R = 1024
    grid = (NP // R,)
    nsteps = NP // R

    def body(p1_ref, p2_ref, ninv_ref, bt_ref,
             gw1_ref, gb1_ref, gw2_ref, gb2_ref,
             mw1_ref, mb1_ref, mw2_ref, mb2_ref,
             nw1_ref, nb1_ref, nw2_ref, nb2_ref,
             h1_ref, h2_ref, g1_ref, g2_ref,
             ps1_acc, ps2_acc, cnt_acc):
        i = pl.program_id(0)
        ninv = ninv_ref[...]
        S1 = p1_ref[0] + p1_ref[1]
        S2 = p2_ref[0] + p2_ref[1]
        z1 = jnp.maximum(
            jnp.dot(S1 * ninv, gw1_ref[...],
                    preferred_element_type=jnp.float32) + gb1_ref[...], 0.0)
        z2 = jnp.maximum(
            jnp.dot(S2 * ninv, gw2_ref[...],
                    preferred_element_type=jnp.float32) + gb2_ref[...], 0.0)
        t1 = jnp.maximum(
            jnp.dot(z1, mw1_ref[...], preferred_element_type=jnp.float32)
            + mb1_ref[...], 0.0)
        t2 = jnp.maximum(
            jnp.dot(z2, mw1_ref[...], preferred_element_type=jnp.float32)
            + mb1_ref[...], 0.0)
        h1_ref[...] = jnp.dot(t1, mw2_ref[...],
                              preferred_element_type=jnp.float32) + mb2_ref[...]
        h2_ref[...] = jnp.dot(t2, mw2_ref[...],
                              preferred_element_type=jnp.float32) + mb2_ref[...]
        rows = i * R + lax.broadcasted_iota(jnp.int32, (R, G), 0)
        valid = (rows < N).astype(jnp.float32)
        oh = (bt_ref[...] == lax.broadcasted_iota(jnp.int32, (R, G), 1))
        oh = oh.astype(jnp.float32)
        dn = (((0,), (0,)), ((), ()))

        @pl.when(i == 0)
        def _():
            ps1_acc[...] = jnp.zeros((G, D), jnp.float32)
            ps2_acc[...] = jnp.zeros((G, D), jnp.float32)
            cnt_acc[...] = jnp.zeros((G, D), jnp.float32)

        ps1_acc[...] += lax.dot_general(oh, z1 * valid, dn,
                                        preferred_element_type=jnp.float32)
        ps2_acc[...] += lax.dot_general(oh, z2 * valid, dn,
                                        preferred_element_type=jnp.float32)
        cnt_acc[...] += lax.dot_general(oh, valid, dn,
                                        preferred_element_type=jnp.float32)

        @pl.when(i == nsteps - 1)
        def _():
            icnt = 1.0 / jnp.maximum(cnt_acc[...], 1.0)
            gp1 = ps1_acc[...] * icnt
            gp2 = ps2_acc[...] * icnt
            u1 = jnp.maximum(
                jnp.dot(gp1, nw1_ref[...], preferred_element_type=jnp.float32)
                + nb1_ref[...], 0.0)
            u2 = jnp.maximum(
                jnp.dot(gp2, nw1_ref[...], preferred_element_type=jnp.float32)
                + nb1_ref[...], 0.0)
            g1_ref[...] = jnp.dot(u1, nw2_ref[...],
                                  preferred_element_type=jnp.float32) + nb2_ref[...]
            g2_ref[...] = jnp.dot(u2, nw2_ref[...],
                                  preferred_element_type=jnp.float32) + nb2_ref[...]

    wspec = pl.BlockSpec((D, D), lambda i: (0, 0))
    bspec = pl.BlockSpec((1, D), lambda i: (0, 0))
    out = pl.pallas_call(
        body,
        grid=grid,
        in_specs=[
            pl.BlockSpec((NC, R, D), lambda i: (0, i, 0)),
            pl.BlockSpec((NC, R, D), lambda i: (0, i, 0)),
            pl.BlockSpec((R, D), lambda i: (i, 0)),
            pl.BlockSpec((R, G), lambda i: (i, 0)),
            wspec, bspec, wspec, bspec,
            wspec, bspec, wspec, bspec,
            wspec, bspec, wspec, bspec,
        ],
        out_specs=[
            pl.BlockSpec((R, D), lambda i: (i, 0)),
            pl.BlockSpec((R, D), lambda i: (i, 0)),
            pl.BlockSpec((G, D), lambda i: (0, 0)),
            pl.BlockSpec((G, D), lambda i: (0, 0)),
        ],
        out_shape=[
            jax.ShapeDtypeStruct((NP, D), jnp.float32),
            jax.ShapeDtypeStruct((NP, D), jnp.float32),
            jax.ShapeDtypeStruct((G, D), jnp.float32),
            jax.ShapeDtypeStruct((G, D), jnp.float32),
        ],
        scratch_shapes=[
            pltpu.VMEM((G, D), jnp.float32),
            pltpu.VMEM((G, D), jnp.float32),
            pltpu.VMEM((G, D), jnp.float32),
        ],
    )(p1, p2, ninv, batch_bc,
      g1W2, g1b2, g2W2, g2b2,
      m1W1, m1b1, m1W2, m1b2,
      m2W1, m2b1, m2W2, m2b2)
    return out


def kernel(x, edge_index, batch,
           g1W1, g1b1, g1W2, g1b2,
           g2W1, g2b1, g2W2, g2b2,
           m1W1, m1b1, m1W2, m1b2,
           m2W1, m2b1, m2W2, m2b2):
    src = edge_index[0].astype(jnp.int32)
    dst = edge_index[1].astype(jnp.int32)
    pad = jnp.full((EP - E,), N, jnp.int32)
    src_t = jnp.concatenate([src, pad]).reshape(NC, NS, CH, CHUNK)
    dst_t = jnp.concatenate([dst, pad]).reshape(NC, NS, CH, CHUNK)

    dwa = D + 16
    xp = jnp.zeros((NP, dwa), jnp.float32)
    xp = xp.at[:N, :D].set(x).at[:N, D].set(1.0)
    zeros_a = jnp.zeros((NP, dwa), jnp.float32)
    zeros_d = jnp.zeros((NP, D), jnp.float32)

    parts0 = _sc_aggregate(xp, src_t, dst_t, zeros_a, dwa)
    h1, h2, ninv = _tc_layer1(parts0, g1W1, g1b1.reshape(1, D),
                              g2W1, g2b1.reshape(1, D))

    parts1 = _sc_aggregate(h1, src_t, dst_t, zeros_d, D)
    parts2 = _sc_aggregate(h2, src_t, dst_t, zeros_d, D)

    batch_bc = jnp.zeros((NP,), jnp.int32).at[:N].set(batch.astype(jnp.int32))
    batch_bc = jnp.broadcast_to(batch_bc[:, None], (NP, G))

    h1o, h2o, g1o, g2o = _tc_layer2_heads(
        parts1, parts2, ninv, batch_bc,
        g1W2, g1b2.reshape(1, D), g2W2, g2b2.reshape(1, D),
        m1W1, m1b1.reshape(1, D), m1W2, m1b2.reshape(1, D),
        m2W1, m2b1.reshape(1, D), m2W2, m2b2.reshape(1, D))

    return h1o[:N], h2o[:N], g1o, g2o


# speedup vs baseline: 2.8400x; 1.1209x over previous
import functools

import jax
import jax.numpy as jnp
from jax import lax
from jax.experimental import pallas as pl
from jax.experimental.pallas import tpu as pltpu
from jax.experimental.pallas import tpu_sc as plsc

N = 10000
E = 320000
D = 128
G = 128

NC = 2
NS = 16
CHUNK = 128
CH = 80
EP = NC * NS * CH * CHUNK
NP = 10240
RPT = NP // NS


def _sc_aggregate(feat, idx_t, zeros, dw):
    mesh = plsc.VectorSubcoreMesh(core_axis_name="c", subcore_axis_name="s")

    @functools.partial(
        pl.kernel,
        out_type=jax.ShapeDtypeStruct((NC, NP, dw), jnp.float32),
        mesh=mesh,
        compiler_params=pltpu.CompilerParams(use_tc_tiling_on_sc=False),
        scratch_types=[
            pltpu.VMEM((2, CHUNK), jnp.int32),
            pltpu.VMEM((2, CHUNK), jnp.int32),
            pltpu.VMEM((CHUNK, dw), jnp.float32),
            pltpu.VMEM((CHUNK, dw), jnp.float32),
            pltpu.VMEM_SHARED((NP, dw), jnp.float32),
            pltpu.SemaphoreType.DMA,
            pltpu.SemaphoreType.DMA,
            pltpu.SemaphoreType.DMA,
        ],
    )
    def agg(feat_hbm, idx_hbm, zer_hbm, out_hbm,
            idx0_v, idx1_v, rows0_v, rows1_v, acc_s,
            isem1, sem0, sem1):
        c = lax.axis_index("c")
        s = lax.axis_index("s")
        pltpu.sync_copy(zer_hbm.at[pl.ds(s * RPT, RPT)],
                        acc_s.at[pl.ds(s * RPT, RPT)])
        plsc.subcore_barrier()

        pltpu.sync_copy(idx_hbm.at[c, s, 0], idx0_v)
        pltpu.async_copy(feat_hbm.at[idx0_v.at[0]], rows0_v, sem0)
        pltpu.async_copy(idx_hbm.at[c, s, 1], idx1_v, isem1)

        @pl.loop(0, CH, step=2)
        def body(j):
            pltpu.make_async_copy(idx_hbm.at[c, s, j + 1], idx1_v,
                                  isem1).wait()
            pltpu.async_copy(feat_hbm.at[idx1_v.at[0]], rows1_v, sem1)
            pltpu.make_async_copy(feat_hbm.at[idx0_v.at[0]], rows0_v,
                                  sem0).wait()
            pltpu.sync_copy(rows0_v, acc_s.at[idx0_v.at[1]], add=True)

            @pl.when(j + 2 < CH)
            def _():
                pltpu.sync_copy(idx_hbm.at[c, s, j + 2], idx0_v)
                pltpu.async_copy(feat_hbm.at[idx0_v.at[0]], rows0_v, sem0)

            pltpu.make_async_copy(feat_hbm.at[idx1_v.at[0]], rows1_v,
                                  sem1).wait()
            pltpu.sync_copy(rows1_v, acc_s.at[idx1_v.at[1]], add=True)

            @pl.when(j + 3 < CH)
            def _():
                pltpu.async_copy(idx_hbm.at[c, s, j + 3], idx1_v, isem1)
        plsc.subcore_barrier()
        pltpu.sync_copy(acc_s.at[pl.ds(s * RPT, RPT)],
                        out_hbm.at[c, pl.ds(s * RPT, RPT)])

    return agg(feat, idx_t, zeros)


def _tc_layer1(parts, g1W1, g1b1, g2W1, g2b1):
    R = 1024
    grid = (NP // R,)
    dwa = D + 16

    def body(p_ref, w1_ref, b1_ref, w2_ref, b2_ref,
             h1_ref, h2_ref, ninv_ref):
        i = pl.program_id(0)
        S = p_ref[0] + p_ref[1]
        deg = S[:, D:D + 1]
        ninv = 1.0 / jnp.maximum(deg, 1.0)
        A = S[:, :D] * ninv
        rows = i * R + lax.broadcasted_iota(jnp.int32, (R, D), 0)
        valid = (rows < N).astype(jnp.float32)
        h1 = jnp.maximum(
            0.8 * jnp.dot(A, w1_ref[...], preferred_element_type=jnp.float32)
            + b1_ref[...], 0.0)
        h2 = jnp.maximum(
            0.9 * jnp.dot(A, w2_ref[...], preferred_element_type=jnp.float32)
            + b2_ref[...], 0.0)
        h1_ref[...] = h1 * valid
        h2_ref[...] = h2 * valid
        ninv_ref[...] = jnp.broadcast_to(ninv, (R, D))

    out = pl.pallas_call(
        body,
        grid=grid,
        in_specs=[
            pl.BlockSpec((NC, R, dwa), lambda i: (0, i, 0)),
            pl.BlockSpec((D, D), lambda i: (0, 0)),
            pl.BlockSpec((1, D), lambda i: (0, 0)),
            pl.BlockSpec((D, D), lambda i: (0, 0)),
            pl.BlockSpec((1, D), lambda i: (0, 0)),
        ],
        out_specs=[
            pl.BlockSpec((R, D), lambda i: (i, 0)),
            pl.BlockSpec((R, D), lambda i: (i, 0)),
            pl.BlockSpec((R, D), lambda i: (i, 0)),
        ],
        out_shape=[
            jax.ShapeDtypeStruct((NP, D), jnp.float32),
            jax.ShapeDtypeStruct((NP, D), jnp.float32),
            jax.ShapeDtypeStruct((NP, D), jnp.float32),
        ],
    )(parts, g1W1, g1b1, g2W1, g2b1)
    return out


def _tc_layer2_heads(p1, p2, ninv, batch_bc,
                     g1W2, g1b2, g2W2, g2b2,
                     m1W1, m1b1, m1W2, m1b2,
                     m2W1, m2b1, m2W2, m2b2):
    R = 1024
    grid = (NP // R,)
    nsteps = NP // R

    def body(p1_ref, p2_ref, ninv_ref, bt_ref,
             gw1_ref, gb1_ref, gw2_ref, gb2_ref,
             mw1_ref, mb1_ref, mw2_ref, mb2_ref,
             nw1_ref, nb1_ref, nw2_ref, nb2_ref,
             h1_ref, h2_ref, g1_ref, g2_ref,
             ps1_acc, ps2_acc, cnt_acc):
        i = pl.program_id(0)
        ninv = ninv_ref[...]
        S1 = p1_ref[0] + p1_ref[1]
        S2 = p2_ref[0] + p2_ref[1]
        z1 = jnp.maximum(
            jnp.dot(S1 * ninv, gw1_ref[...],
                    preferred_element_type=jnp.float32) + gb1_ref[...], 0.0)
        z2 = jnp.maximum(
            jnp.dot(S2 * ninv, gw2_ref[...],
                    preferred_element_type=jnp.float32) + gb2_ref[...], 0.0)
        t1 = jnp.maximum(
            jnp.dot(z1, mw1_ref[...], preferred_element_type=jnp.float32)
            + mb1_ref[...], 0.0)
        t2 = jnp.maximum(
            jnp.dot(z2, mw1_ref[...], preferred_element_type=jnp.float32)
            + mb1_ref[...], 0.0)
        h1_ref[...] = jnp.dot(t1, mw2_ref[...],
                              preferred_element_type=jnp.float32) + mb2_ref[...]
        h2_ref[...] = jnp.dot(t2, mw2_ref[...],
                              preferred_element_type=jnp.float32) + mb2_ref[...]
        rows = i * R + lax.broadcasted_iota(jnp.int32, (R, G), 0)
        valid = (rows < N).astype(jnp.float32)
        oh = (bt_ref[...] == lax.broadcasted_iota(jnp.int32, (R, G), 1))
        oh = oh.astype(jnp.float32)
        dn = (((0,), (0,)), ((), ()))

        @pl.when(i == 0)
        def _():
            ps1_acc[...] = jnp.zeros((G, D), jnp.float32)
            ps2_acc[...] = jnp.zeros((G, D), jnp.float32)
            cnt_acc[...] = jnp.zeros((G, D), jnp.float32)

        ps1_acc[...] += lax.dot_general(oh, z1 * valid, dn,
                                        preferred_element_type=jnp.float32)
        ps2_acc[...] += lax.dot_general(oh, z2 * valid, dn,
                                        preferred_element_type=jnp.float32)
        cnt_acc[...] += lax.dot_general(oh, valid, dn,
                                        preferred_element_type=jnp.float32)

        @pl.when(i == nsteps - 1)
        def _():
            icnt = 1.0 / jnp.maximum(cnt_acc[...], 1.0)
            gp1 = ps1_acc[...] * icnt
            gp2 = ps2_acc[...] * icnt
            u1 = jnp.maximum(
                jnp.dot(gp1, nw1_ref[...], preferred_element_type=jnp.float32)
                + nb1_ref[...], 0.0)
            u2 = jnp.maximum(
                jnp.dot(gp2, nw1_ref[...], preferred_element_type=jnp.float32)
                + nb1_ref[...], 0.0)
            g1_ref[...] = jnp.dot(u1, nw2_ref[...],
                                  preferred_element_type=jnp.float32) + nb2_ref[...]
            g2_ref[...] = jnp.dot(u2, nw2_ref[...],
                                  preferred_element_type=jnp.float32) + nb2_ref[...]

    wspec = pl.BlockSpec((D, D), lambda i: (0, 0))
    bspec = pl.BlockSpec((1, D), lambda i: (0, 0))
    out = pl.pallas_call(
        body,
        grid=grid,
        in_specs=[
            pl.BlockSpec((NC, R, D), lambda i: (0, i, 0)),
            pl.BlockSpec((NC, R, D), lambda i: (0, i, 0)),
            pl.BlockSpec((R, D), lambda i: (i, 0)),
            pl.BlockSpec((R, G), lambda i: (i, 0)),
            wspec, bspec, wspec, bspec,
            wspec, bspec, wspec, bspec,
            wspec, bspec, wspec, bspec,
        ],
        out_specs=[
            pl.BlockSpec((R, D), lambda i: (i, 0)),
            pl.BlockSpec((R, D), lambda i: (i, 0)),
            pl.BlockSpec((G, D), lambda i: (0, 0)),
            pl.BlockSpec((G, D), lambda i: (0, 0)),
        ],
        out_shape=[
            jax.ShapeDtypeStruct((NP, D), jnp.float32),
            jax.ShapeDtypeStruct((NP, D), jnp.float32),
            jax.ShapeDtypeStruct((G, D), jnp.float32),
            jax.ShapeDtypeStruct((G, D), jnp.float32),
        ],
        scratch_shapes=[
            pltpu.VMEM((G, D), jnp.float32),
            pltpu.VMEM((G, D), jnp.float32),
            pltpu.VMEM((G, D), jnp.float32),
        ],
    )(p1, p2, ninv, batch_bc,
      g1W2, g1b2, g2W2, g2b2,
      m1W1, m1b1, m1W2, m1b2,
      m2W1, m2b1, m2W2, m2b2)
    return out


def kernel(x, edge_index, batch,
           g1W1, g1b1, g1W2, g1b2,
           g2W1, g2b1, g2W2, g2b2,
           m1W1, m1b1, m1W2, m1b2,
           m2W1, m2b1, m2W2, m2b2):
    src = edge_index[0].astype(jnp.int32)
    dst = edge_index[1].astype(jnp.int32)
    pad = jnp.full((EP - E,), N, jnp.int32)
    src_t = jnp.concatenate([src, pad]).reshape(NC, NS, CH, CHUNK)
    dst_t = jnp.concatenate([dst, pad]).reshape(NC, NS, CH, CHUNK)
    idx_t = jnp.stack([src_t, dst_t], axis=3)

    dwa = D + 16
    xp = jnp.zeros((NP, dwa), jnp.float32)
    xp = xp.at[:N, :D].set(x).at[:N, D].set(1.0)
    zeros_a = jnp.zeros((NP, dwa), jnp.float32)
    zeros_d = jnp.zeros((NP, D), jnp.float32)

    parts0 = _sc_aggregate(xp, idx_t, zeros_a, dwa)
    h1, h2, ninv = _tc_layer1(parts0, g1W1, g1b1.reshape(1, D),
                              g2W1, g2b1.reshape(1, D))

    parts1 = _sc_aggregate(h1, idx_t, zeros_d, D)
    parts2 = _sc_aggregate(h2, idx_t, zeros_d, D)

    batch_bc = jnp.zeros((NP,), jnp.int32).at[:N].set(batch.astype(jnp.int32))
    batch_bc = jnp.broadcast_to(batch_bc[:, None], (NP, G))

    h1o, h2o, g1o, g2o = _tc_layer2_heads(
        parts1, parts2, ninv, batch_bc,
        g1W2, g1b2.reshape(1, D), g2W2, g2b2.reshape(1, D),
        m1W1, m1b1.reshape(1, D), m1W2, m1b2.reshape(1, D),
        m2W1, m2b1.reshape(1, D), m2W2, m2b2.reshape(1, D))

    return h1o[:N], h2o[:N], g1o, g2o


# speedup vs baseline: 8.1524x; 2.8705x over previous
import functools

import jax
import jax.numpy as jnp
from jax import lax
from jax.experimental import pallas as pl
from jax.experimental.pallas import tpu as pltpu
from jax.experimental.pallas import tpu_sc as plsc

N = 10000
E = 320000
D = 128
G = 128

NC = 2
NS = 16
CHUNK = 128
CH = 80
EP = NC * NS * CH * CHUNK
NP = 10240
RPT = NP // NS


def _sc_aggregate(feat, idx_t, zeros, dw):
    mesh = plsc.VectorSubcoreMesh(core_axis_name="c", subcore_axis_name="s")

    @functools.partial(
        pl.kernel,
        out_type=jax.ShapeDtypeStruct((NC, NP, dw), jnp.float32),
        mesh=mesh,
        compiler_params=pltpu.CompilerParams(use_tc_tiling_on_sc=False),
        scratch_types=[
            pltpu.VMEM((2, CHUNK), jnp.int32),
            pltpu.VMEM((2, CHUNK), jnp.int32),
            pltpu.VMEM((CHUNK, dw), jnp.float32),
            pltpu.VMEM((CHUNK, dw), jnp.float32),
            pltpu.VMEM_SHARED((NP, dw), jnp.float32),
            pltpu.SemaphoreType.DMA,
            pltpu.SemaphoreType.DMA,
            pltpu.SemaphoreType.DMA,
        ],
    )
    def agg(feat_hbm, idx_hbm, zer_hbm, out_hbm,
            idx0_v, idx1_v, rows0_v, rows1_v, acc_s,
            isem1, sem0, sem1):
        c = lax.axis_index("c")
        s = lax.axis_index("s")
        pltpu.sync_copy(zer_hbm.at[pl.ds(s * RPT, RPT)],
                        acc_s.at[pl.ds(s * RPT, RPT)])
        plsc.subcore_barrier()

        pltpu.sync_copy(idx_hbm.at[c, s, 0], idx0_v)
        pltpu.async_copy(feat_hbm.at[idx0_v.at[0]], rows0_v, sem0)
        pltpu.async_copy(idx_hbm.at[c, s, 1], idx1_v, isem1)

        @pl.loop(0, CH, step=2)
        def body(j):
            pltpu.make_async_copy(idx_hbm.at[c, s, j + 1], idx1_v,
                                  isem1).wait()
            pltpu.async_copy(feat_hbm.at[idx1_v.at[0]], rows1_v, sem1)
            pltpu.make_async_copy(feat_hbm.at[idx0_v.at[0]], rows0_v,
                                  sem0).wait()
            pltpu.sync_copy(rows0_v, acc_s.at[idx0_v.at[1]], add=True)

            @pl.when(j + 2 < CH)
            def _():
                pltpu.sync_copy(idx_hbm.at[c, s, j + 2], idx0_v)
                pltpu.async_copy(feat_hbm.at[idx0_v.at[0]], rows0_v, sem0)

            pltpu.make_async_copy(feat_hbm.at[idx1_v.at[0]], rows1_v,
                                  sem1).wait()
            pltpu.sync_copy(rows1_v, acc_s.at[idx1_v.at[1]], add=True)

            @pl.when(j + 3 < CH)
            def _():
                pltpu.async_copy(idx_hbm.at[c, s, j + 3], idx1_v, isem1)
        plsc.subcore_barrier()
        pltpu.sync_copy(acc_s.at[pl.ds(s * RPT, RPT)],
                        out_hbm.at[c, pl.ds(s * RPT, RPT)])

    return agg(feat, idx_t, zeros)


def _tc_layer1(parts, g1W1, g1b1, g2W1, g2b1):
    R = 1024
    grid = (NP // R,)
    dwa = D + 16

    def body(p_ref, w1_ref, b1_ref, w2_ref, b2_ref,
             h1_ref, h2_ref, ninv_ref):
        i = pl.program_id(0)
        S = p_ref[0] + p_ref[1]
        deg = S[:, D:D + 1]
        ninv = 1.0 / jnp.maximum(deg, 1.0)
        A = S[:, :D] * ninv
        rows = i * R + lax.broadcasted_iota(jnp.int32, (R, D), 0)
        valid = (rows < N).astype(jnp.float32)
        h1 = jnp.maximum(
            0.8 * jnp.dot(A, w1_ref[...], preferred_element_type=jnp.float32)
            + b1_ref[...], 0.0)
        h2 = jnp.maximum(
            0.9 * jnp.dot(A, w2_ref[...], preferred_element_type=jnp.float32)
            + b2_ref[...], 0.0)
        h1_ref[...] = h1 * valid
        h2_ref[...] = h2 * valid
        ninv_ref[...] = jnp.broadcast_to(ninv, (R, D))

    out = pl.pallas_call(
        body,
        grid=grid,
        in_specs=[
            pl.BlockSpec((NC, R, dwa), lambda i: (0, i, 0)),
            pl.BlockSpec((D, D), lambda i: (0, 0)),
            pl.BlockSpec((1, D), lambda i: (0, 0)),
            pl.BlockSpec((D, D), lambda i: (0, 0)),
            pl.BlockSpec((1, D), lambda i: (0, 0)),
        ],
        out_specs=[
            pl.BlockSpec((R, D), lambda i: (i, 0)),
            pl.BlockSpec((R, D), lambda i: (i, 0)),
            pl.BlockSpec((R, D), lambda i: (i, 0)),
        ],
        out_shape=[
            jax.ShapeDtypeStruct((NP, D), jnp.float32),
            jax.ShapeDtypeStruct((NP, D), jnp.float32),
            jax.ShapeDtypeStruct((NP, D), jnp.float32),
        ],
    )(parts, g1W1, g1b1, g2W1, g2b1)
    return out


def _tc_layer2_heads(p1, p2, ninv, batch_bc,
                     g1W2, g1b2, g2W2, g2b2,
                     m1W1, m1b1, m1W2, m1b2,
                     m2W1, m2b1, m2W2, m2b2):
    R = 1024
    grid = (NP // R,)
    nsteps = NP // R

    def body(p1_ref, p2_ref, ninv_ref, bt_ref,
             gw1_ref, gb1_ref, gw2_ref, gb2_ref,
             mw1_ref, mb1_ref, mw2_ref, mb2_ref,
             nw1_ref, nb1_ref, nw2_ref, nb2_ref,
             h1_ref, h2_ref, g1_ref, g2_ref,
             ps1_acc, ps2_acc, cnt_acc):
        i = pl.program_id(0)
        ninv = ninv_ref[...]
        S1 = p1_ref[0] + p1_ref[1]
        S2 = p2_ref[0] + p2_ref[1]
        z1 = jnp.maximum(
            jnp.dot(S1 * ninv, gw1_ref[...],
                    preferred_element_type=jnp.float32) + gb1_ref[...], 0.0)
        z2 = jnp.maximum(
            jnp.dot(S2 * ninv, gw2_ref[...],
                    preferred_element_type=jnp.float32) + gb2_ref[...], 0.0)
        t1 = jnp.maximum(
            jnp.dot(z1, mw1_ref[...], preferred_element_type=jnp.float32)
            + mb1_ref[...], 0.0)
        t2 = jnp.maximum(
            jnp.dot(z2, mw1_ref[...], preferred_element_type=jnp.float32)
            + mb1_ref[...], 0.0)
        h1_ref[...] = jnp.dot(t1, mw2_ref[...],
                              preferred_element_type=jnp.float32) + mb2_ref[...]
        h2_ref[...] = jnp.dot(t2, mw2_ref[...],
                              preferred_element_type=jnp.float32) + mb2_ref[...]
        rows = i * R + lax.broadcasted_iota(jnp.int32, (R, G), 0)
        valid = (rows < N).astype(jnp.float32)
        oh = (bt_ref[...] == lax.broadcasted_iota(jnp.int32, (R, G), 1))
        oh = oh.astype(jnp.float32)
        dn = (((0,), (0,)), ((), ()))

        @pl.when(i == 0)
        def _():
            ps1_acc[...] = jnp.zeros((G, D), jnp.float32)
            ps2_acc[...] = jnp.zeros((G, D), jnp.float32)
            cnt_acc[...] = jnp.zeros((G, D), jnp.float32)

        ps1_acc[...] += lax.dot_general(oh, z1 * valid, dn,
                                        preferred_element_type=jnp.float32)
        ps2_acc[...] += lax.dot_general(oh, z2 * valid, dn,
                                        preferred_element_type=jnp.float32)
        cnt_acc[...] += lax.dot_general(oh, valid, dn,
                                        preferred_element_type=jnp.float32)

        @pl.when(i == nsteps - 1)
        def _():
            icnt = 1.0 / jnp.maximum(cnt_acc[...], 1.0)
            gp1 = ps1_acc[...] * icnt
            gp2 = ps2_acc[...] * icnt
            u1 = jnp.maximum(
                jnp.dot(gp1, nw1_ref[...], preferred_element_type=jnp.float32)
                + nb1_ref[...], 0.0)
            u2 = jnp.maximum(
                jnp.dot(gp2, nw1_ref[...], preferred_element_type=jnp.float32)
                + nb1_ref[...], 0.0)
            g1_ref[...] = jnp.dot(u1, nw2_ref[...],
                                  preferred_element_type=jnp.float32) + nb2_ref[...]
            g2_ref[...] = jnp.dot(u2, nw2_ref[...],
                                  preferred_element_type=jnp.float32) + nb2_ref[...]

    wspec = pl.BlockSpec((D, D), lambda i: (0, 0))
    bspec = pl.BlockSpec((1, D), lambda i: (0, 0))
    out = pl.pallas_call(
        body,
        grid=grid,
        in_specs=[
            pl.BlockSpec((NC, R, D), lambda i: (0, i, 0)),
            pl.BlockSpec((NC, R, D), lambda i: (0, i, 0)),
            pl.BlockSpec((R, D), lambda i: (i, 0)),
            pl.BlockSpec((R, G), lambda i: (i, 0)),
            wspec, bspec, wspec, bspec,
            wspec, bspec, wspec, bspec,
            wspec, bspec, wspec, bspec,
        ],
        out_specs=[
            pl.BlockSpec((R, D), lambda i: (i, 0)),
            pl.BlockSpec((R, D), lambda i: (i, 0)),
            pl.BlockSpec((G, D), lambda i: (0, 0)),
            pl.BlockSpec((G, D), lambda i: (0, 0)),
        ],
        out_shape=[
            jax.ShapeDtypeStruct((NP, D), jnp.float32),
            jax.ShapeDtypeStruct((NP, D), jnp.float32),
            jax.ShapeDtypeStruct((G, D), jnp.float32),
            jax.ShapeDtypeStruct((G, D), jnp.float32),
        ],
        scratch_shapes=[
            pltpu.VMEM((G, D), jnp.float32),
            pltpu.VMEM((G, D), jnp.float32),
            pltpu.VMEM((G, D), jnp.float32),
        ],
    )(p1, p2, ninv, batch_bc,
      g1W2, g1b2, g2W2, g2b2,
      m1W1, m1b1, m1W2, m1b2,
      m2W1, m2b1, m2W2, m2b2)
    return out


def kernel(x, edge_index, batch,
           g1W1, g1b1, g1W2, g1b2,
           g2W1, g2b1, g2W2, g2b2,
           m1W1, m1b1, m1W2, m1b2,
           m2W1, m2b1, m2W2, m2b2):
    ntiles = NC * NS
    ppt = (EP - E) // ntiles
    src = edge_index[0].astype(jnp.int32).reshape(ntiles, E // ntiles)
    dst = edge_index[1].astype(jnp.int32).reshape(ntiles, E // ntiles)
    pad = jnp.broadcast_to(N + jnp.arange(ppt, dtype=jnp.int32) % (NP - N),
                           (ntiles, ppt))
    src_t = jnp.concatenate([src, pad], axis=1).reshape(NC, NS, CH, CHUNK)
    dst_t = jnp.concatenate([dst, pad], axis=1).reshape(NC, NS, CH, CHUNK)
    idx_t = jnp.stack([src_t, dst_t], axis=3)

    dwa = D + 16
    xp = jnp.zeros((NP, dwa), jnp.float32)
    xp = xp.at[:N, :D].set(x).at[:N, D].set(1.0)
    zeros_a = jnp.zeros((NP, dwa), jnp.float32)
    zeros_d = jnp.zeros((NP, D), jnp.float32)

    parts0 = _sc_aggregate(xp, idx_t, zeros_a, dwa)
    h1, h2, ninv = _tc_layer1(parts0, g1W1, g1b1.reshape(1, D),
                              g2W1, g2b1.reshape(1, D))

    parts1 = _sc_aggregate(h1, idx_t, zeros_d, D)
    parts2 = _sc_aggregate(h2, idx_t, zeros_d, D)

    batch_bc = jnp.zeros((NP,), jnp.int32).at[:N].set(batch.astype(jnp.int32))
    batch_bc = jnp.broadcast_to(batch_bc[:, None], (NP, G))

    h1o, h2o, g1o, g2o = _tc_layer2_heads(
        parts1, parts2, ninv, batch_bc,
        g1W2, g1b2.reshape(1, D), g2W2, g2b2.reshape(1, D),
        m1W1, m1b1.reshape(1, D), m1W2, m1b2.reshape(1, D),
        m2W1, m2b1.reshape(1, D), m2W2, m2b2.reshape(1, D))

    return h1o[:N], h2o[:N], g1o, g2o


# speedup vs baseline: 8.6875x; 1.0656x over previous
import functools

import jax
import jax.numpy as jnp
from jax import lax
from jax.experimental import pallas as pl
from jax.experimental.pallas import tpu as pltpu
from jax.experimental.pallas import tpu_sc as plsc

N = 10000
E = 320000
D = 128
G = 128
DEGW = 16

NC = 2
NS = 16
EPT = E // (NC * NS)
CHUNK = 128
CHF = EPT // CHUNK
TAIL = EPT - CHF * CHUNK
RPT = N // NS


def _sc_aggregate(feat, src, dst, zeros, ones=None, zeros16=None):
    with_deg = ones is not None
    mesh = plsc.VectorSubcoreMesh(core_axis_name="c", subcore_axis_name="s")

    out_type = [jax.ShapeDtypeStruct((NC, N, D), jnp.float32)]
    scratch = [
        pltpu.VMEM((2, CHUNK), jnp.int32),
        pltpu.VMEM((2, CHUNK), jnp.int32),
        pltpu.VMEM((2, TAIL), jnp.int32),
        pltpu.VMEM((CHUNK, D), jnp.float32),
        pltpu.VMEM((CHUNK, D), jnp.float32),
        pltpu.VMEM((TAIL, D), jnp.float32),
        pltpu.VMEM_SHARED((N, D), jnp.float32),
        pltpu.SemaphoreType.DMA,
        pltpu.SemaphoreType.DMA,
        pltpu.SemaphoreType.DMA,
    ]
    if with_deg:
        out_type.append(jax.ShapeDtypeStruct((NC, N, DEGW), jnp.float32))
        scratch.append(pltpu.VMEM((CHUNK, DEGW), jnp.float32))
        scratch.append(pltpu.VMEM_SHARED((N, DEGW), jnp.float32))

    @functools.partial(
        pl.kernel,
        out_type=out_type,
        mesh=mesh,
        compiler_params=pltpu.CompilerParams(use_tc_tiling_on_sc=False),
        scratch_types=scratch,
    )
    def agg(*refs):
        if with_deg:
            (feat_hbm, src_hbm, dst_hbm, zer_hbm, ones_hbm, zer16_hbm,
             out_hbm, deg_hbm,
             idx0_v, idx1_v, idxt_v, rows0_v, rows1_v, rowst_v, acc_s,
             isem1, sem0, sem1, ones_v, acc16_s) = refs
        else:
            (feat_hbm, src_hbm, dst_hbm, zer_hbm,
             out_hbm,
             idx0_v, idx1_v, idxt_v, rows0_v, rows1_v, rowst_v, acc_s,
             isem1, sem0, sem1) = refs
        c = lax.axis_index("c")
        s = lax.axis_index("s")
        base = (c * NS + s) * EPT
        pltpu.sync_copy(zer_hbm.at[pl.ds(s * RPT, RPT)],
                        acc_s.at[pl.ds(s * RPT, RPT)])
        if with_deg:
            pltpu.sync_copy(zer16_hbm.at[pl.ds(s * RPT, RPT)],
                            acc16_s.at[pl.ds(s * RPT, RPT)])
            pltpu.sync_copy(ones_hbm, ones_v)
        plsc.subcore_barrier()

        def idx_load_async(j, ibuf, sem):
            off = base + j * CHUNK
            pltpu.async_copy(src_hbm.at[pl.ds(off, CHUNK)], ibuf.at[0], sem)
            pltpu.async_copy(dst_hbm.at[pl.ds(off, CHUNK)], ibuf.at[1], sem)

        def idx_load_sync(j, ibuf):
            off = base + j * CHUNK
            pltpu.sync_copy(src_hbm.at[pl.ds(off, CHUNK)], ibuf.at[0])
            pltpu.sync_copy(dst_hbm.at[pl.ds(off, CHUNK)], ibuf.at[1])

        def idx_wait(j, ibuf, sem):
            off = base + j * CHUNK
            pltpu.make_async_copy(src_hbm.at[pl.ds(off, CHUNK)], ibuf.at[0],
                                  sem).wait()
            pltpu.make_async_copy(dst_hbm.at[pl.ds(off, CHUNK)], ibuf.at[1],
                                  sem).wait()

        def scatter(ibuf, rbuf):
            pltpu.sync_copy(rbuf, acc_s.at[ibuf.at[1]], add=True)
            if with_deg:
                pltpu.sync_copy(ones_v, acc16_s.at[ibuf.at[1]], add=True)

        idx_load_sync(0, idx0_v)
        pltpu.async_copy(feat_hbm.at[idx0_v.at[0]], rows0_v, sem0)
        idx_load_async(1, idx1_v, isem1)

        @pl.loop(0, CHF, step=2)
        def body(j):
            idx_wait(j + 1, idx1_v, isem1)
            pltpu.async_copy(feat_hbm.at[idx1_v.at[0]], rows1_v, sem1)
            pltpu.make_async_copy(feat_hbm.at[idx0_v.at[0]], rows0_v,
                                  sem0).wait()
            scatter(idx0_v, rows0_v)

            @pl.when(j + 2 < CHF)
            def _():
                idx_load_sync(j + 2, idx0_v)
                pltpu.async_copy(feat_hbm.at[idx0_v.at[0]], rows0_v, sem0)

            pltpu.make_async_copy(feat_hbm.at[idx1_v.at[0]], rows1_v,
                                  sem1).wait()
            scatter(idx1_v, rows1_v)

            @pl.when(j + 3 < CHF)
            def _():
                idx_load_async(j + 3, idx1_v, isem1)

        toff = base + CHF * CHUNK
        pltpu.sync_copy(src_hbm.at[pl.ds(toff, TAIL)], idxt_v.at[0])
        pltpu.sync_copy(dst_hbm.at[pl.ds(toff, TAIL)], idxt_v.at[1])
        pltpu.async_copy(feat_hbm.at[idxt_v.at[0]], rowst_v, sem0).wait()
        pltpu.sync_copy(rowst_v, acc_s.at[idxt_v.at[1]], add=True)
        if with_deg:
            pltpu.sync_copy(ones_v.at[pl.ds(0, TAIL)],
                            acc16_s.at[idxt_v.at[1]], add=True)

        plsc.subcore_barrier()
        pltpu.sync_copy(acc_s.at[pl.ds(s * RPT, RPT)],
                        out_hbm.at[c, pl.ds(s * RPT, RPT)])
        if with_deg:
            pltpu.sync_copy(acc16_s.at[pl.ds(s * RPT, RPT)],
                            deg_hbm.at[c, pl.ds(s * RPT, RPT)])

    if with_deg:
        return agg(feat, src, dst, zeros, ones, zeros16)
    return agg(feat, src, dst, zeros)


def _tc_layer1(parts, dparts, g1W1, g1b1, g2W1, g2b1):
    R = 2000
    grid = (N // R,)

    def body(p_ref, d_ref, w1_ref, b1_ref, w2_ref, b2_ref, h1_ref, h2_ref):
        S = p_ref[0] + p_ref[1]
        deg = d_ref[0][:, :1] + d_ref[1][:, :1]
        A = S * (1.0 / jnp.maximum(deg, 1.0))
        h1_ref[...] = jnp.maximum(
            0.8 * jnp.dot(A, w1_ref[...], preferred_element_type=jnp.float32)
            + b1_ref[...], 0.0)
        h2_ref[...] = jnp.maximum(
            0.9 * jnp.dot(A, w2_ref[...], preferred_element_type=jnp.float32)
            + b2_ref[...], 0.0)

    return pl.pallas_call(
        body,
        grid=grid,
        in_specs=[
            pl.BlockSpec((NC, R, D), lambda i: (0, i, 0)),
            pl.BlockSpec((NC, R, DEGW), lambda i: (0, i, 0)),
            pl.BlockSpec((D, D), lambda i: (0, 0)),
            pl.BlockSpec((1, D), lambda i: (0, 0)),
            pl.BlockSpec((D, D), lambda i: (0, 0)),
            pl.BlockSpec((1, D), lambda i: (0, 0)),
        ],
        out_specs=[
            pl.BlockSpec((R, D), lambda i: (i, 0)),
            pl.BlockSpec((R, D), lambda i: (i, 0)),
        ],
        out_shape=[
            jax.ShapeDtypeStruct((N, D), jnp.float32),
            jax.ShapeDtypeStruct((N, D), jnp.float32),
        ],
    )(parts, dparts, g1W1, g1b1, g2W1, g2b1)


def _tc_layer2_heads(p1, p2, dparts, batch_bc,
                     g1W2, g1b2, g2W2, g2b2,
                     m1W1, m1b1, m1W2, m1b2,
                     m2W1, m2b1, m2W2, m2b2):
    R = 2000
    grid = (N // R,)
    nsteps = N // R

    def body(p1_ref, p2_ref, d_ref, bt_ref,
             gw1_ref, gb1_ref, gw2_ref, gb2_ref,
             mw1_ref, mb1_ref, mw2_ref, mb2_ref,
             nw1_ref, nb1_ref, nw2_ref, nb2_ref,
             h1_ref, h2_ref, g1_ref, g2_ref,
             ps1_acc, ps2_acc, cnt_acc):
        i = pl.program_id(0)
        deg = d_ref[0][:, :1] + d_ref[1][:, :1]
        ninv = 1.0 / jnp.maximum(deg, 1.0)
        z1 = jnp.maximum(
            jnp.dot((p1_ref[0] + p1_ref[1]) * ninv, gw1_ref[...],
                    preferred_element_type=jnp.float32) + gb1_ref[...], 0.0)
        z2 = jnp.maximum(
            jnp.dot((p2_ref[0] + p2_ref[1]) * ninv, gw2_ref[...],
                    preferred_element_type=jnp.float32) + gb2_ref[...], 0.0)
        t1 = jnp.maximum(
            jnp.dot(z1, mw1_ref[...], preferred_element_type=jnp.float32)
            + mb1_ref[...], 0.0)
        t2 = jnp.maximum(
            jnp.dot(z2, mw1_ref[...], preferred_element_type=jnp.float32)
            + mb1_ref[...], 0.0)
        h1_ref[...] = jnp.dot(t1, mw2_ref[...],
                              preferred_element_type=jnp.float32) + mb2_ref[...]
        h2_ref[...] = jnp.dot(t2, mw2_ref[...],
                              preferred_element_type=jnp.float32) + mb2_ref[...]
        oh = (bt_ref[...] == lax.broadcasted_iota(jnp.int32, (R, G), 1))
        oh = oh.astype(jnp.float32)
        dn = (((0,), (0,)), ((), ()))

        @pl.when(i == 0)
        def _():
            ps1_acc[...] = jnp.zeros((G, D), jnp.float32)
            ps2_acc[...] = jnp.zeros((G, D), jnp.float32)
            cnt_acc[...] = jnp.zeros((G, D), jnp.float32)

        ps1_acc[...] += lax.dot_general(oh, z1, dn,
                                        preferred_element_type=jnp.float32)
        ps2_acc[...] += lax.dot_general(oh, z2, dn,
                                        preferred_element_type=jnp.float32)
        cnt_acc[...] += lax.dot_general(oh, jnp.ones((R, D), jnp.float32), dn,
                                        preferred_element_type=jnp.float32)

        @pl.when(i == nsteps - 1)
        def _():
            icnt = 1.0 / jnp.maximum(cnt_acc[...], 1.0)
            u1 = jnp.maximum(
                jnp.dot(ps1_acc[...] * icnt, nw1_ref[...],
                        preferred_element_type=jnp.float32) + nb1_ref[...], 0.0)
            u2 = jnp.maximum(
                jnp.dot(ps2_acc[...] * icnt, nw1_ref[...],
                        preferred_element_type=jnp.float32) + nb1_ref[...], 0.0)
            g1_ref[...] = jnp.dot(u1, nw2_ref[...],
                                  preferred_element_type=jnp.float32) + nb2_ref[...]
            g2_ref[...] = jnp.dot(u2, nw2_ref[...],
                                  preferred_element_type=jnp.float32) + nb2_ref[...]

    wspec = pl.BlockSpec((D, D), lambda i: (0, 0))
    bspec = pl.BlockSpec((1, D), lambda i: (0, 0))
    return pl.pallas_call(
        body,
        grid=grid,
        in_specs=[
            pl.BlockSpec((NC, R, D), lambda i: (0, i, 0)),
            pl.BlockSpec((NC, R, D), lambda i: (0, i, 0)),
            pl.BlockSpec((NC, R, DEGW), lambda i: (0, i, 0)),
            pl.BlockSpec((R, G), lambda i: (i, 0)),
            wspec, bspec, wspec, bspec,
            wspec, bspec, wspec, bspec,
            wspec, bspec, wspec, bspec,
        ],
        out_specs=[
            pl.BlockSpec((R, D), lambda i: (i, 0)),
            pl.BlockSpec((R, D), lambda i: (i, 0)),
            pl.BlockSpec((G, D), lambda i: (0, 0)),
            pl.BlockSpec((G, D), lambda i: (0, 0)),
        ],
        out_shape=[
            jax.ShapeDtypeStruct((N, D), jnp.float32),
            jax.ShapeDtypeStruct((N, D), jnp.float32),
            jax.ShapeDtypeStruct((G, D), jnp.float32),
            jax.ShapeDtypeStruct((G, D), jnp.float32),
        ],
        scratch_shapes=[
            pltpu.VMEM((G, D), jnp.float32),
            pltpu.VMEM((G, D), jnp.float32),
            pltpu.VMEM((G, D), jnp.float32),
        ],
    )(p1, p2, dparts, batch_bc,
      g1W2, g1b2, g2W2, g2b2,
      m1W1, m1b1, m1W2, m1b2,
      m2W1, m2b1, m2W2, m2b2)


def kernel(x, edge_index, batch,
           g1W1, g1b1, g1W2, g1b2,
           g2W1, g2b1, g2W2, g2b2,
           m1W1, m1b1, m1W2, m1b2,
           m2W1, m2b1, m2W2, m2b2):
    src = edge_index[0].astype(jnp.int32)
    dst = edge_index[1].astype(jnp.int32)
    zeros = jnp.zeros((N, D), jnp.float32)
    zeros16 = jnp.zeros((N, DEGW), jnp.float32)
    ones = jnp.ones((CHUNK, DEGW), jnp.float32)

    parts0, dparts = _sc_aggregate(x, src, dst, zeros, ones, zeros16)
    h1, h2 = _tc_layer1(parts0, dparts, g1W1, g1b1.reshape(1, D),
                        g2W1, g2b1.reshape(1, D))

    parts1 = _sc_aggregate(h1, src, dst, zeros)[0]
    parts2 = _sc_aggregate(h2, src, dst, zeros)[0]

    batch_bc = jnp.broadcast_to(batch.astype(jnp.int32)[:, None], (N, G))

    return _tc_layer2_heads(
        parts1, parts2, dparts, batch_bc,
        g1W2, g1b2.reshape(1, D), g2W2, g2b2.reshape(1, D),
        m1W1, m1b1.reshape(1, D), m1W2, m1b2.reshape(1, D),
        m2W1, m2b1.reshape(1, D), m2W2, m2b2.reshape(1, D))


# speedup vs baseline: 8.9418x; 1.0293x over previous
import functools

import jax
import jax.numpy as jnp
from jax import lax
from jax.experimental import pallas as pl
from jax.experimental.pallas import tpu as pltpu
from jax.experimental.pallas import tpu_sc as plsc

N = 10000
E = 320000
D = 128
G = 128
DEGW = 16

NC = 2
NS = 16
EPT = E // (NC * NS)
CHUNK = 128
CHF = EPT // CHUNK
TAIL = EPT - CHF * CHUNK
RPT = N // NS


def _sc_aggregate(feat, ei, with_deg):
    mesh = plsc.VectorSubcoreMesh(core_axis_name="c", subcore_axis_name="s")

    out_type = [jax.ShapeDtypeStruct((NC, N, D), jnp.float32)]
    scratch = [
        pltpu.VMEM((2, CHUNK), jnp.int32),
        pltpu.VMEM((2, CHUNK), jnp.int32),
        pltpu.VMEM((2, TAIL), jnp.int32),
        pltpu.VMEM((CHUNK, D), jnp.float32),
        pltpu.VMEM((CHUNK, D), jnp.float32),
        pltpu.VMEM((TAIL, D), jnp.float32),
        pltpu.VMEM_SHARED((N, D), jnp.float32),
        pltpu.SemaphoreType.DMA,
        pltpu.SemaphoreType.DMA,
        pltpu.SemaphoreType.DMA,
    ]
    if with_deg:
        out_type.append(jax.ShapeDtypeStruct((NC, N, DEGW), jnp.float32))
        scratch.append(pltpu.VMEM((CHUNK, DEGW), jnp.float32))
        scratch.append(pltpu.VMEM((RPT // 5, DEGW), jnp.float32))
        scratch.append(pltpu.VMEM_SHARED((N, DEGW), jnp.float32))

    @functools.partial(
        pl.kernel,
        out_type=out_type,
        mesh=mesh,
        compiler_params=pltpu.CompilerParams(use_tc_tiling_on_sc=False),
        scratch_types=scratch,
    )
    def agg(*refs):
        if with_deg:
            (feat_hbm, ei_hbm,
             out_hbm, deg_hbm,
             idx0_v, idx1_v, idxt_v, rows0_v, rows1_v, rowst_v, acc_s,
             isem1, sem0, sem1, ones_v, z16_v, acc16_s) = refs
        else:
            (feat_hbm, ei_hbm,
             out_hbm,
             idx0_v, idx1_v, idxt_v, rows0_v, rows1_v, rowst_v, acc_s,
             isem1, sem0, sem1) = refs
        c = lax.axis_index("c")
        s = lax.axis_index("s")
        base = (c * NS + s) * EPT
        zv = jnp.zeros((16,), jnp.float32)

        @pl.loop(0, RPT // 5)
        def zrow(r):
            for k in range(D // 16):
                rows0_v[r, pl.ds(k * 16, 16)] = zv

        for t in range(5):
            pltpu.sync_copy(rows0_v.at[pl.ds(0, RPT // 5)],
                            acc_s.at[pl.ds(s * RPT + t * (RPT // 5),
                                           RPT // 5)])
        if with_deg:
            ov = zv + 1.0

            @pl.loop(0, CHUNK)
            def fill_ones(k):
                ones_v[k, pl.ds(0, DEGW)] = ov

            @pl.loop(0, RPT // 5)
            def z16row(r):
                z16_v[r, pl.ds(0, DEGW)] = zv

            for t in range(5):
                pltpu.sync_copy(
                    z16_v,
                    acc16_s.at[pl.ds(s * RPT + t * (RPT // 5), RPT // 5)])
        plsc.subcore_barrier()

        def idx_load_async(j, ibuf, sem):
            off = base + j * CHUNK
            pltpu.async_copy(ei_hbm.at[0, pl.ds(off, CHUNK)], ibuf.at[0], sem)
            pltpu.async_copy(ei_hbm.at[1, pl.ds(off, CHUNK)], ibuf.at[1], sem)

        def idx_load_sync(j, ibuf):
            off = base + j * CHUNK
            pltpu.sync_copy(ei_hbm.at[0, pl.ds(off, CHUNK)], ibuf.at[0])
            pltpu.sync_copy(ei_hbm.at[1, pl.ds(off, CHUNK)], ibuf.at[1])

        def idx_wait(j, ibuf, sem):
            off = base + j * CHUNK
            pltpu.make_async_copy(ei_hbm.at[0, pl.ds(off, CHUNK)],
                                  ibuf.at[0], sem).wait()
            pltpu.make_async_copy(ei_hbm.at[1, pl.ds(off, CHUNK)],
                                  ibuf.at[1], sem).wait()

        def scatter(ibuf, rbuf):
            pltpu.sync_copy(rbuf, acc_s.at[ibuf.at[1]], add=True)
            if with_deg:
                pltpu.sync_copy(ones_v, acc16_s.at[ibuf.at[1]], add=True)

        idx_load_sync(0, idx0_v)
        pltpu.async_copy(feat_hbm.at[idx0_v.at[0]], rows0_v, sem0)
        idx_load_async(1, idx1_v, isem1)

        @pl.loop(0, CHF, step=2)
        def body(j):
            idx_wait(j + 1, idx1_v, isem1)
            pltpu.async_copy(feat_hbm.at[idx1_v.at[0]], rows1_v, sem1)
            pltpu.make_async_copy(feat_hbm.at[idx0_v.at[0]], rows0_v,
                                  sem0).wait()
            scatter(idx0_v, rows0_v)

            @pl.when(j + 2 < CHF)
            def _():
                idx_load_sync(j + 2, idx0_v)
                pltpu.async_copy(feat_hbm.at[idx0_v.at[0]], rows0_v, sem0)

            pltpu.make_async_copy(feat_hbm.at[idx1_v.at[0]], rows1_v,
                                  sem1).wait()
            scatter(idx1_v, rows1_v)

            @pl.when(j + 3 < CHF)
            def _():
                idx_load_async(j + 3, idx1_v, isem1)

        toff = base + CHF * CHUNK
        pltpu.sync_copy(ei_hbm.at[0, pl.ds(toff, TAIL)], idxt_v.at[0])
        pltpu.sync_copy(ei_hbm.at[1, pl.ds(toff, TAIL)], idxt_v.at[1])
        pltpu.async_copy(feat_hbm.at[idxt_v.at[0]], rowst_v, sem0).wait()
        pltpu.sync_copy(rowst_v, acc_s.at[idxt_v.at[1]], add=True)
        if with_deg:
            pltpu.sync_copy(ones_v.at[pl.ds(0, TAIL)],
                            acc16_s.at[idxt_v.at[1]], add=True)

        plsc.subcore_barrier()
        pltpu.sync_copy(acc_s.at[pl.ds(s * RPT, RPT)],
                        out_hbm.at[c, pl.ds(s * RPT, RPT)])
        if with_deg:
            pltpu.sync_copy(acc16_s.at[pl.ds(s * RPT, RPT)],
                            deg_hbm.at[c, pl.ds(s * RPT, RPT)])

    return agg(feat, ei)


def _tc_layer1(parts, ninv_bc, g1W1, g1b1, g2W1, g2b1):
    R = 2000
    grid = (N // R,)

    def body(p_ref, d_ref, w1_ref, b1_ref, w2_ref, b2_ref, h1_ref, h2_ref):
        A = (p_ref[0] + p_ref[1]) * d_ref[...]
        h1_ref[...] = jnp.maximum(
            0.8 * jnp.dot(A, w1_ref[...], preferred_element_type=jnp.float32)
            + b1_ref[...], 0.0)
        h2_ref[...] = jnp.maximum(
            0.9 * jnp.dot(A, w2_ref[...], preferred_element_type=jnp.float32)
            + b2_ref[...], 0.0)

    return pl.pallas_call(
        body,
        grid=grid,
        in_specs=[
            pl.BlockSpec((NC, R, D), lambda i: (0, i, 0)),
            pl.BlockSpec((R, D), lambda i: (i, 0)),
            pl.BlockSpec((D, D), lambda i: (0, 0)),
            pl.BlockSpec((1, D), lambda i: (0, 0)),
            pl.BlockSpec((D, D), lambda i: (0, 0)),
            pl.BlockSpec((1, D), lambda i: (0, 0)),
        ],
        out_specs=[
            pl.BlockSpec((R, D), lambda i: (i, 0)),
            pl.BlockSpec((R, D), lambda i: (i, 0)),
        ],
        out_shape=[
            jax.ShapeDtypeStruct((N, D), jnp.float32),
            jax.ShapeDtypeStruct((N, D), jnp.float32),
        ],
    )(parts, ninv_bc, g1W1, g1b1, g2W1, g2b1)


def _tc_layer2_heads(p1, p2, ninv_bc, batch_bc,
                     g1W2, g1b2, g2W2, g2b2,
                     m1W1, m1b1, m1W2, m1b2,
                     m2W1, m2b1, m2W2, m2b2):
    R = 2000
    grid = (N // R,)
    nsteps = N // R

    def body(p1_ref, p2_ref, d_ref, bt_ref,
             gw1_ref, gb1_ref, gw2_ref, gb2_ref,
             mw1_ref, mb1_ref, mw2_ref, mb2_ref,
             nw1_ref, nb1_ref, nw2_ref, nb2_ref,
             h1_ref, h2_ref, g1_ref, g2_ref,
             ps1_acc, ps2_acc, cnt_acc):
        i = pl.program_id(0)
        ninv = d_ref[...]
        z1 = jnp.maximum(
            jnp.dot((p1_ref[0] + p1_ref[1]) * ninv, gw1_ref[...],
                    preferred_element_type=jnp.float32) + gb1_ref[...], 0.0)
        z2 = jnp.maximum(
            jnp.dot((p2_ref[0] + p2_ref[1]) * ninv, gw2_ref[...],
                    preferred_element_type=jnp.float32) + gb2_ref[...], 0.0)
        t1 = jnp.maximum(
            jnp.dot(z1, mw1_ref[...], preferred_element_type=jnp.float32)
            + mb1_ref[...], 0.0)
        t2 = jnp.maximum(
            jnp.dot(z2, mw1_ref[...], preferred_element_type=jnp.float32)
            + mb1_ref[...], 0.0)
        h1_ref[...] = jnp.dot(t1, mw2_ref[...],
                              preferred_element_type=jnp.float32) + mb2_ref[...]
        h2_ref[...] = jnp.dot(t2, mw2_ref[...],
                              preferred_element_type=jnp.float32) + mb2_ref[...]
        oh = (bt_ref[...] == lax.broadcasted_iota(jnp.int32, (R, G), 1))
        oh = oh.astype(jnp.float32)
        dn = (((0,), (0,)), ((), ()))

        @pl.when(i == 0)
        def _():
            ps1_acc[...] = jnp.zeros((G, D), jnp.float32)
            ps2_acc[...] = jnp.zeros((G, D), jnp.float32)
            cnt_acc[...] = jnp.zeros((G, D), jnp.float32)

        ps1_acc[...] += lax.dot_general(oh, z1, dn,
                                        preferred_element_type=jnp.float32)
        ps2_acc[...] += lax.dot_general(oh, z2, dn,
                                        preferred_element_type=jnp.float32)
        cnt_acc[...] += lax.dot_general(oh, jnp.ones((R, D), jnp.float32), dn,
                                        preferred_element_type=jnp.float32)

        @pl.when(i == nsteps - 1)
        def _():
            icnt = 1.0 / jnp.maximum(cnt_acc[...], 1.0)
            u1 = jnp.maximum(
                jnp.dot(ps1_acc[...] * icnt, nw1_ref[...],
                        preferred_element_type=jnp.float32) + nb1_ref[...], 0.0)
            u2 = jnp.maximum(
                jnp.dot(ps2_acc[...] * icnt, nw1_ref[...],
                        preferred_element_type=jnp.float32) + nb1_ref[...], 0.0)
            g1_ref[...] = jnp.dot(u1, nw2_ref[...],
                                  preferred_element_type=jnp.float32) + nb2_ref[...]
            g2_ref[...] = jnp.dot(u2, nw2_ref[...],
                                  preferred_element_type=jnp.float32) + nb2_ref[...]

    wspec = pl.BlockSpec((D, D), lambda i: (0, 0))
    bspec = pl.BlockSpec((1, D), lambda i: (0, 0))
    return pl.pallas_call(
        body,
        grid=grid,
        in_specs=[
            pl.BlockSpec((NC, R, D), lambda i: (0, i, 0)),
            pl.BlockSpec((NC, R, D), lambda i: (0, i, 0)),
            pl.BlockSpec((R, D), lambda i: (i, 0)),
            pl.BlockSpec((R, G), lambda i: (i, 0)),
            wspec, bspec, wspec, bspec,
            wspec, bspec, wspec, bspec,
            wspec, bspec, wspec, bspec,
        ],
        out_specs=[
            pl.BlockSpec((R, D), lambda i: (i, 0)),
            pl.BlockSpec((R, D), lambda i: (i, 0)),
            pl.BlockSpec((G, D), lambda i: (0, 0)),
            pl.BlockSpec((G, D), lambda i: (0, 0)),
        ],
        out_shape=[
            jax.ShapeDtypeStruct((N, D), jnp.float32),
            jax.ShapeDtypeStruct((N, D), jnp.float32),
            jax.ShapeDtypeStruct((G, D), jnp.float32),
            jax.ShapeDtypeStruct((G, D), jnp.float32),
        ],
        scratch_shapes=[
            pltpu.VMEM((G, D), jnp.float32),
            pltpu.VMEM((G, D), jnp.float32),
            pltpu.VMEM((G, D), jnp.float32),
        ],
    )(p1, p2, ninv_bc, batch_bc,
      g1W2, g1b2, g2W2, g2b2,
      m1W1, m1b1, m1W2, m1b2,
      m2W1, m2b1, m2W2, m2b2)


def kernel(x, edge_index, batch,
           g1W1, g1b1, g1W2, g1b2,
           g2W1, g2b1, g2W2, g2b2,
           m1W1, m1b1, m1W2, m1b2,
           m2W1, m2b1, m2W2, m2b2):
    ei = edge_index.astype(jnp.int32)

    parts0, dparts = _sc_aggregate(x, ei, True)
    deg = dparts[0, :, 0] + dparts[1, :, 0]
    ninv_bc = jnp.broadcast_to(
        (1.0 / jnp.maximum(deg, 1.0))[:, None], (N, D))

    h1, h2 = _tc_layer1(parts0, ninv_bc, g1W1, g1b1.reshape(1, D),
                        g2W1, g2b1.reshape(1, D))

    parts1 = _sc_aggregate(h1, ei, False)[0]
    parts2 = _sc_aggregate(h2, ei, False)[0]

    batch_bc = jnp.broadcast_to(batch.astype(jnp.int32)[:, None], (N, G))

    return _tc_layer2_heads(
        parts1, parts2, ninv_bc, batch_bc,
        g1W2, g1b2.reshape(1, D), g2W2, g2b2.reshape(1, D),
        m1W1, m1b1.reshape(1, D), m1W2, m1b2.reshape(1, D),
        m2W1, m2b1.reshape(1, D), m2W2, m2b2.reshape(1, D))
